# baseline (speedup 1.0000x reference)
import functools
import jax
import jax.numpy as jnp
from jax import lax
from jax.experimental import pallas as pl
from jax.experimental.pallas import tpu as pltpu
from jax.experimental.pallas import tpu_sc as plsc

_K = 64
_N = 32768
_ROWS = 128
_L = 16
_NV = _N // _L
_CAP = 4096
_NC = 2
_NS = 16
_NW = _NC * _NS
_RPW = _ROWS // _NW
_GUESS0 = 0x40000000
_MARGIN = 1 << 21
_IMIN = -0x80000000


def _skey(v):
    s = lax.bitcast_convert_type(v, jnp.int32)
    return s ^ (lax.shift_right_arithmetic(s, 31) & jnp.int32(0x7FFFFFFF))


def _sc_body(x_hbm, o_hbm, b0_v, b1_v, out_v, candk_v, candi_v,
             ki0_v, ki1_v, ls0, ls1, osem):
    wid = lax.axis_index("s") * _NC + lax.axis_index("c")
    lane = lax.broadcasted_iota(jnp.int32, (_L,), 0)
    zero_v = jnp.zeros((_L,), jnp.int32)
    fzero_v = jnp.zeros((_L,), jnp.float32)
    neg1_v = jnp.full((_L,), -1, jnp.int32)
    bufs = [b0_v, b1_v]
    lsems = [ls0, ls1]
    kbufs = [ki0_v, ki1_v]

    @plsc.parallel_loop(0, _NV, unroll=8)
    def _(i):
        out_v[pl.ds(pl.multiple_of(i * _L, 8), _L)] = fzero_v

    def start_load(rr):
        return pltpu.async_copy(x_hbm.at[wid * _RPW + rr], bufs[rr % 2],
                                lsems[rr % 2])

    ld = [None] * _RPW
    for rr in range(min(2, _RPW)):
        ld[rr] = start_load(rr)
    odma = [None] * _RPW

    t_spec = jnp.int32(_GUESS0)
    for rr in range(_RPW):
        row_v = bufs[rr % 2]
        ld[rr].wait()

        def not_ok(carry):
            _, _, _, c, fin = carry
            bad = (c < _K) | (c > _CAP)
            return bad & jnp.logical_not(fin)

        def attempt(carry):
            t_prev, lo, hi, c, fin = carry
            ran = c >= 0
            lo = jnp.where(ran & (c > _CAP), t_prev, lo)
            hi = jnp.where(ran & (c < _K), t_prev, hi)
            ulo = lax.bitcast_convert_type(lo, jnp.uint32) ^ jnp.uint32(0x80000000)
            uhi = lax.bitcast_convert_type(hi, jnp.uint32) ^ jnp.uint32(0x80000000)
            umid = ulo + lax.shift_right_logical(uhi - ulo, jnp.uint32(1))
            mid = lax.bitcast_convert_type(umid ^ jnp.uint32(0x80000000), jnp.int32)
            narrow = (uhi - ulo) <= jnp.uint32(1)
            fin = ran & narrow
            t_s = jnp.where(ran, jnp.where(narrow, lo, mid), t_prev)
            tsu = lax.bitcast_convert_type(t_s, jnp.uint32)

            @plsc.parallel_loop(0, _NV, unroll=16, carry=(neg1_v, lane))
            def comp_out(i, cr):
                offm1, idxv = cr
                v = row_v[pl.ds(pl.multiple_of(i * _L, 8), _L)]
                key = _skey(v)
                m = key >= t_s
                du = lax.bitcast_convert_type(key, jnp.uint32) - tsu
                ci = jnp.cumsum(m.astype(jnp.int32))
                pos = offm1 + ci
                okm = m & (pos < _CAP)
                plsc.store_scatter(candk_v, [pos],
                                   lax.bitcast_convert_type(du, jnp.int32),
                                   mask=okm)
                plsc.store_scatter(candi_v, [pos], idxv, mask=okm)
                return (offm1 + plsc.all_reduce_population_count(m), idxv + _L)

            c_new = jnp.max(comp_out[0]) + jnp.int32(1)
            return (t_s, lo, hi, c_new, fin)

        t_s, _, _, c_tot, _ = lax.while_loop(
            not_ok, attempt,
            (t_spec, jnp.int32(_IMIN), jnp.int32(0x7FFFFFFF),
             jnp.int32(-1), jnp.bool_(False)))

        c_eff = jnp.minimum(c_tot, jnp.int32(_CAP))
        tpos = c_eff + lane
        okp = tpos < _CAP
        tposc = jnp.minimum(tpos, jnp.int32(_CAP - 1))
        plsc.store_scatter(candk_v, [tposc], zero_v, mask=okp)
        plsc.store_scatter(candi_v, [tposc], jnp.full((_L,), _N, jnp.int32),
                           mask=okp)
        nv = (c_eff + jnp.int32(_L - 1)) // _L

        @plsc.parallel_loop(0, nv, unroll=2, carry=zero_v)
        def mx_out(j, a):
            ck = lax.bitcast_convert_type(
                candk_v[pl.ds(pl.multiple_of(j * _L, 8), _L)], jnp.uint32)
            return jnp.maximum(a, lax.bitcast_convert_type(
                lax.shift_right_logical(ck, jnp.uint32(1)), jnp.int32))
        mx = jnp.max(mx_out)
        mxf = lax.bitcast_convert_type(
            lax.convert_element_type(mx, jnp.float32), jnp.int32)
        b0 = jnp.clip(lax.shift_right_logical(mxf, 23) - jnp.int32(125),
                      jnp.int32(1), jnp.int32(32))

        def bit_step(b_, t):
            b = (b0 - jnp.int32(1) - b_).astype(jnp.uint32)
            cand = t | lax.shift_left(jnp.uint32(1), b)

            @plsc.parallel_loop(0, nv, unroll=2, carry=zero_v)
            def acc_out(j, a):
                ck = lax.bitcast_convert_type(
                    candk_v[pl.ds(pl.multiple_of(j * _L, 8), _L)], jnp.uint32)
                return a + (ck >= cand).astype(jnp.int32)
            c = jnp.sum(acc_out)
            return jnp.where(c >= _K, cand, t)
        t_du = lax.fori_loop(0, b0, bit_step, jnp.uint32(0))

        def cnt2(j, carry):
            a_ge, a_gt = carry
            ck = lax.bitcast_convert_type(
                candk_v[pl.ds(pl.multiple_of(j * _L, 8), _L)], jnp.uint32)
            return (a_ge + (ck >= t_du).astype(jnp.int32),
                    a_gt + (ck > t_du).astype(jnp.int32))
        a_ge, a_gt = lax.fori_loop(0, nv, cnt2, (zero_v, zero_v))
        n_ge = jnp.sum(a_ge)
        r = jnp.int32(_K) - jnp.sum(a_gt)

        def tie(_):
            def jb(b_, J):
                b = jnp.int32(14) - b_
                cand = J | lax.shift_left(jnp.int32(1), b)
                def cnt(j, a):
                    o = pl.multiple_of(j * _L, 8)
                    ck = lax.bitcast_convert_type(candk_v[pl.ds(o, _L)],
                                                  jnp.uint32)
                    civ = candi_v[pl.ds(o, _L)]
                    m = (ck == t_du) & (civ < cand)
                    return a + m.astype(jnp.int32)
                c = jnp.sum(lax.fori_loop(0, nv, cnt, zero_v))
                return jnp.where(c < r, cand, J)
            return lax.fori_loop(0, 15, jb, jnp.int32(0))
        J = lax.cond(n_ge == _K, lambda _: jnp.int32(_N - 1), tie,
                     jnp.int32(0))

        if rr >= 1:
            odma[rr - 1].wait()
            kprev = kbufs[(rr - 1) % 2]
            for j in range(_K // _L):
                zi = kprev[pl.ds(j * _L, _L)]
                plsc.store_scatter(out_v, [zi], fzero_v)

        kcur = kbufs[rr % 2]

        @plsc.parallel_loop(0, nv, unroll=2, carry=neg1_v)
        def kc_out(j, offm1):
            o = pl.multiple_of(j * _L, 8)
            ck = lax.bitcast_convert_type(candk_v[pl.ds(o, _L)], jnp.uint32)
            civ = candi_v[pl.ds(o, _L)]
            m = (ck > t_du) | ((ck == t_du) & (civ <= J))
            ci = jnp.cumsum(m.astype(jnp.int32))
            pos = offm1 + ci
            okm = m & (pos < _K)
            cv = plsc.load_gather(row_v, [jnp.where(okm, civ, 0)])
            plsc.store_scatter(out_v, [civ], cv, mask=okm)
            plsc.store_scatter(kcur, [pos], civ, mask=okm)
            return offm1 + plsc.all_reduce_population_count(m)

        odma[rr] = pltpu.async_copy(out_v, o_hbm.at[wid * _RPW + rr], osem)

        t_key = lax.bitcast_convert_type(
            lax.bitcast_convert_type(t_s, jnp.uint32) + t_du, jnp.int32)
        t_spec = jnp.where(t_key < jnp.int32(_IMIN + _MARGIN),
                           jnp.int32(_IMIN), t_key - jnp.int32(_MARGIN))
        if rr + 2 < _RPW:
            ld[rr + 2] = start_load(rr + 2)

    odma[_RPW - 1].wait()


def kernel(x):
    mesh = plsc.VectorSubcoreMesh(core_axis_name="c", subcore_axis_name="s")
    fn = functools.partial(
        pl.kernel,
        mesh=mesh,
        compiler_params=pltpu.CompilerParams(needs_layout_passes=False),
        out_type=jax.ShapeDtypeStruct((_ROWS, _N), jnp.float32),
        scratch_types=[
            pltpu.VMEM((_N,), jnp.float32),
            pltpu.VMEM((_N,), jnp.float32),
            pltpu.VMEM((_N,), jnp.float32),
            pltpu.VMEM((_CAP,), jnp.int32),
            pltpu.VMEM((_CAP,), jnp.int32),
            pltpu.VMEM((_K,), jnp.int32),
            pltpu.VMEM((_K,), jnp.int32),
            pltpu.SemaphoreType.DMA,
            pltpu.SemaphoreType.DMA,
            pltpu.SemaphoreType.DMA,
        ],
    )(_sc_body)
    return fn(x)


# speedup vs baseline: 1.1162x; 1.1162x over previous
import functools
import jax
import jax.numpy as jnp
from jax import lax
from jax.experimental import pallas as pl
from jax.experimental.pallas import tpu as pltpu
from jax.experimental.pallas import tpu_sc as plsc

_K = 64
_N = 32768
_ROWS = 128
_L = 16
_NV = _N // _L
_CAP = 4096
_NC = 2
_NS = 16
_NW = _NC * _NS
_RPW = _ROWS // _NW
_GUESS0 = 0x40000000
_MARGIN = 1 << 21
_IMIN = -0x80000000


def _skey(v):
    s = lax.bitcast_convert_type(v, jnp.int32)
    return s ^ (lax.shift_right_arithmetic(s, 31) & jnp.int32(0x7FFFFFFF))


def _sc_body(x_hbm, o_hbm, b0_v, b1_v, out_v, candk_v, candi_v,
             ki0_v, ki1_v, ls0, ls1, osem):
    wid = lax.axis_index("s") * _NC + lax.axis_index("c")
    lane = lax.broadcasted_iota(jnp.int32, (_L,), 0)
    zero_v = jnp.zeros((_L,), jnp.int32)
    fzero_v = jnp.zeros((_L,), jnp.float32)
    neg1_v = jnp.full((_L,), -1, jnp.int32)
    bufs = [b0_v, b1_v]
    lsems = [ls0, ls1]
    kbufs = [ki0_v, ki1_v]

    @plsc.parallel_loop(0, _NV, unroll=8)
    def _(i):
        out_v[pl.ds(pl.multiple_of(i * _L, 8), _L)] = fzero_v

    def start_load(rr):
        return pltpu.async_copy(x_hbm.at[wid * _RPW + rr], bufs[rr % 2],
                                lsems[rr % 2])

    ld = [None] * _RPW
    for rr in range(min(2, _RPW)):
        ld[rr] = start_load(rr)
    odma = [None] * _RPW

    t_spec = jnp.int32(_GUESS0)
    for rr in range(_RPW):
        row_v = bufs[rr % 2]
        ld[rr].wait()

        def not_ok(carry):
            _, _, _, c, fin = carry
            bad = (c < _K) | (c > _CAP)
            return bad & jnp.logical_not(fin)

        def attempt(carry):
            t_prev, lo, hi, c, fin = carry
            ran = c >= 0
            lo = jnp.where(ran & (c > _CAP), t_prev, lo)
            hi = jnp.where(ran & (c < _K), t_prev, hi)
            ulo = lax.bitcast_convert_type(lo, jnp.uint32) ^ jnp.uint32(0x80000000)
            uhi = lax.bitcast_convert_type(hi, jnp.uint32) ^ jnp.uint32(0x80000000)
            umid = ulo + lax.shift_right_logical(uhi - ulo, jnp.uint32(1))
            mid = lax.bitcast_convert_type(umid ^ jnp.uint32(0x80000000), jnp.int32)
            narrow = (uhi - ulo) <= jnp.uint32(1)
            fin = ran & narrow
            t_s = jnp.where(ran, jnp.where(narrow, lo, mid), t_prev)
            tsu = lax.bitcast_convert_type(t_s, jnp.uint32)

            @plsc.parallel_loop(0, _NV, unroll=8, carry=(neg1_v, lane))
            def comp_out(i, cr):
                offm1, idxv = cr
                v = row_v[pl.ds(pl.multiple_of(i * _L, 8), _L)]
                key = _skey(v)
                m = key >= t_s
                du = lax.bitcast_convert_type(key, jnp.uint32) - tsu
                ci = jnp.cumsum(m.astype(jnp.int32))
                pos = offm1 + ci
                okm = m & (pos < _CAP)
                plsc.store_scatter(candk_v, [pos],
                                   lax.bitcast_convert_type(du, jnp.int32),
                                   mask=okm)
                plsc.store_scatter(candi_v, [pos], idxv, mask=okm)
                return (offm1 + plsc.all_reduce_population_count(m), idxv + _L)

            c_new = jnp.max(comp_out[0]) + jnp.int32(1)
            return (t_s, lo, hi, c_new, fin)

        t_s, _, _, c_tot, _ = lax.while_loop(
            not_ok, attempt,
            (t_spec, jnp.int32(_IMIN), jnp.int32(0x7FFFFFFF),
             jnp.int32(-1), jnp.bool_(False)))

        c_eff = jnp.minimum(c_tot, jnp.int32(_CAP))
        tpos = c_eff + lane
        okp = tpos < _CAP
        tposc = jnp.minimum(tpos, jnp.int32(_CAP - 1))
        plsc.store_scatter(candk_v, [tposc], zero_v, mask=okp)
        plsc.store_scatter(candi_v, [tposc], jnp.full((_L,), _N, jnp.int32),
                           mask=okp)
        nv = (c_eff + jnp.int32(_L - 1)) // _L

        @plsc.parallel_loop(0, nv, unroll=2, carry=zero_v)
        def mx_out(j, a):
            ck = lax.bitcast_convert_type(
                candk_v[pl.ds(pl.multiple_of(j * _L, 8), _L)], jnp.uint32)
            return jnp.maximum(a, lax.bitcast_convert_type(
                lax.shift_right_logical(ck, jnp.uint32(1)), jnp.int32))
        mx = jnp.max(mx_out)
        mxf = lax.bitcast_convert_type(
            lax.convert_element_type(mx, jnp.float32), jnp.int32)
        b0 = jnp.clip(lax.shift_right_logical(mxf, 23) - jnp.int32(125),
                      jnp.int32(1), jnp.int32(32))

        def bit_step(b_, t):
            b = (b0 - jnp.int32(1) - b_).astype(jnp.uint32)
            cand = t | lax.shift_left(jnp.uint32(1), b)

            @plsc.parallel_loop(0, nv, unroll=2, carry=zero_v)
            def acc_out(j, a):
                ck = lax.bitcast_convert_type(
                    candk_v[pl.ds(pl.multiple_of(j * _L, 8), _L)], jnp.uint32)
                return a + (ck >= cand).astype(jnp.int32)
            c = jnp.sum(acc_out)
            return jnp.where(c >= _K, cand, t)
        t_du = lax.fori_loop(0, b0, bit_step, jnp.uint32(0))

        def cnt2(j, carry):
            a_ge, a_gt = carry
            ck = lax.bitcast_convert_type(
                candk_v[pl.ds(pl.multiple_of(j * _L, 8), _L)], jnp.uint32)
            return (a_ge + (ck >= t_du).astype(jnp.int32),
                    a_gt + (ck > t_du).astype(jnp.int32))
        a_ge, a_gt = lax.fori_loop(0, nv, cnt2, (zero_v, zero_v))
        n_ge = jnp.sum(a_ge)
        r = jnp.int32(_K) - jnp.sum(a_gt)

        def tie(_):
            def jb(b_, J):
                b = jnp.int32(14) - b_
                cand = J | lax.shift_left(jnp.int32(1), b)
                def cnt(j, a):
                    o = pl.multiple_of(j * _L, 8)
                    ck = lax.bitcast_convert_type(candk_v[pl.ds(o, _L)],
                                                  jnp.uint32)
                    civ = candi_v[pl.ds(o, _L)]
                    m = (ck == t_du) & (civ < cand)
                    return a + m.astype(jnp.int32)
                c = jnp.sum(lax.fori_loop(0, nv, cnt, zero_v))
                return jnp.where(c < r, cand, J)
            return lax.fori_loop(0, 15, jb, jnp.int32(0))
        J = lax.cond(n_ge == _K, lambda _: jnp.int32(_N - 1), tie,
                     jnp.int32(0))

        if rr >= 1:
            odma[rr - 1].wait()
            kprev = kbufs[(rr - 1) % 2]
            for j in range(_K // _L):
                zi = kprev[pl.ds(j * _L, _L)]
                plsc.store_scatter(out_v, [zi], fzero_v)

        kcur = kbufs[rr % 2]

        @plsc.parallel_loop(0, nv, unroll=2, carry=neg1_v)
        def kc_out(j, offm1):
            o = pl.multiple_of(j * _L, 8)
            ck = lax.bitcast_convert_type(candk_v[pl.ds(o, _L)], jnp.uint32)
            civ = candi_v[pl.ds(o, _L)]
            m = (ck > t_du) | ((ck == t_du) & (civ <= J))
            ci = jnp.cumsum(m.astype(jnp.int32))
            pos = offm1 + ci
            okm = m & (pos < _K)
            cv = plsc.load_gather(row_v, [jnp.where(okm, civ, 0)])
            plsc.store_scatter(out_v, [civ], cv, mask=okm)
            plsc.store_scatter(kcur, [pos], civ, mask=okm)
            return offm1 + plsc.all_reduce_population_count(m)

        odma[rr] = pltpu.async_copy(out_v, o_hbm.at[wid * _RPW + rr], osem)

        t_key = lax.bitcast_convert_type(
            lax.bitcast_convert_type(t_s, jnp.uint32) + t_du, jnp.int32)
        t_spec = jnp.where(t_key < jnp.int32(_IMIN + _MARGIN),
                           jnp.int32(_IMIN), t_key - jnp.int32(_MARGIN))
        if rr + 2 < _RPW:
            ld[rr + 2] = start_load(rr + 2)

    odma[_RPW - 1].wait()


def kernel(x):
    mesh = plsc.VectorSubcoreMesh(core_axis_name="c", subcore_axis_name="s")
    fn = functools.partial(
        pl.kernel,
        mesh=mesh,
        compiler_params=pltpu.CompilerParams(needs_layout_passes=False),
        out_type=jax.ShapeDtypeStruct((_ROWS, _N), jnp.float32),
        scratch_types=[
            pltpu.VMEM((_N,), jnp.float32),
            pltpu.VMEM((_N,), jnp.float32),
            pltpu.VMEM((_N,), jnp.float32),
            pltpu.VMEM((_CAP,), jnp.int32),
            pltpu.VMEM((_CAP,), jnp.int32),
            pltpu.VMEM((_K,), jnp.int32),
            pltpu.VMEM((_K,), jnp.int32),
            pltpu.SemaphoreType.DMA,
            pltpu.SemaphoreType.DMA,
            pltpu.SemaphoreType.DMA,
        ],
    )(_sc_body)
    return fn(x)
